# baseline (speedup 1.0000x reference)
import functools

import jax
import jax.numpy as jnp
from jax import lax
from jax.experimental import pallas as pl
from jax.experimental.pallas import tpu as pltpu
from jax.experimental.pallas import tpu_sc as plsc

T, D, F, E, K = 32, 1024, 704, 64, 8


def _combine_sc(se_hbm, rw_hbm, out_hbm, se_v, rw_v, row_v):
    wid = lax.axis_index("s") * 2 + lax.axis_index("c")
    pltpu.sync_copy(se_hbm, se_v)
    pltpu.sync_copy(rw_hbm, rw_v)
    for j in range(2):
        e = wid * 2 + j
        for half in range(2):
            acc = jnp.zeros((16,), jnp.float32)
            for k in range(K):
                sev = se_v[k, pl.ds(half * 16, 16)]
                rwv = rw_v[k, pl.ds(half * 16, 16)]
                acc = acc + jnp.where(sev == e, rwv, 0.0)
            row_v[pl.ds(half * 16, 16)] = acc
        pltpu.sync_copy(row_v, out_hbm.at[e])


def _moe_kernel(cw_ref, s0_ref, s1_ref, s2_ref, x_ref,
                w0_ref, w1_ref, w2t_ref, o_ref):
    e = pl.program_id(0)
    x = x_ref[...].astype(jnp.bfloat16)
    dn_t = (((1,), (1,)), ((), ()))
    dn_n = (((1,), (0,)), ((), ()))
    w0e = w0_ref[0].astype(jnp.bfloat16)
    w1e = w1_ref[0].astype(jnp.bfloat16)
    w2e = w2t_ref[0].astype(jnp.bfloat16)
    g = jax.lax.dot_general(x, w0e, dn_t, preferred_element_type=jnp.float32)
    g = g * s0_ref[e]
    u = jax.lax.dot_general(x, w1e, dn_t, preferred_element_type=jnp.float32)
    h = ((g * jax.nn.sigmoid(g)) * u).astype(jnp.bfloat16)
    y = jax.lax.dot_general(h, w2e, dn_n, preferred_element_type=jnp.float32)
    cw = cw_ref[0]
    contrib = y * (cw * (s1_ref[e] * s2_ref[e]))

    @pl.when(e == 0)
    def _():
        o_ref[...] = contrib

    @pl.when(e != 0)
    def _():
        o_ref[...] += contrib


def kernel(x, w0, w1, w2, s0, s1, s2, selected_experts, routing_weights,
           gathered_experts_out_buf, select_experts_middle, routing_weights_middle,
           gather_buffer, scatter_buffer, use_ppl):
    se_t = jnp.swapaxes(selected_experts.astype(jnp.int32), 0, 1)
    rw_t = jnp.swapaxes(routing_weights, 0, 1)
    w2t = jnp.swapaxes(w2, 1, 2)

    mesh = plsc.VectorSubcoreMesh(core_axis_name="c", subcore_axis_name="s")
    combT = functools.partial(
        pl.kernel, mesh=mesh,
        out_type=jax.ShapeDtypeStruct((E, T), jnp.float32),
        scratch_types=[
            pltpu.VMEM((K, T), jnp.int32),
            pltpu.VMEM((K, T), jnp.float32),
            pltpu.VMEM((T,), jnp.float32),
        ],
    )(_combine_sc)(se_t, rw_t)
    cw3 = combT.reshape(E, T, 1)

    out = pl.pallas_call(
        _moe_kernel,
        grid=(E,),
        in_specs=[
            pl.BlockSpec((1, T, 1), lambda e: (e, 0, 0)),
            pl.BlockSpec(memory_space=pltpu.SMEM),
            pl.BlockSpec(memory_space=pltpu.SMEM),
            pl.BlockSpec(memory_space=pltpu.SMEM),
            pl.BlockSpec((T, D), lambda e: (0, 0)),
            pl.BlockSpec((1, F, D), lambda e: (e, 0, 0)),
            pl.BlockSpec((1, F, D), lambda e: (e, 0, 0)),
            pl.BlockSpec((1, F, D), lambda e: (e, 0, 0)),
        ],
        out_specs=pl.BlockSpec((T, D), lambda e: (0, 0)),
        out_shape=jax.ShapeDtypeStruct((T, D), jnp.float32),
    )(cw3, s0, s1, s2, x, w0, w1, w2t)
    return out


# speedup vs baseline: 1.1606x; 1.1606x over previous
import jax
import jax.numpy as jnp
from jax.experimental import pallas as pl
from jax.experimental.pallas import tpu as pltpu

T, D, F, E, K = 32, 1024, 704, 64, 8


def _moe_kernel(set_ref, rwt_ref, s0_ref, s1_ref, s2_ref, x_ref,
                w0_ref, w1_ref, w2t_ref, o_ref, se_v, rw_v):
    e = pl.program_id(0)

    @pl.when(e == 0)
    def _():
        se_v[...] = jnp.transpose(set_ref[...])
        rw_v[...] = jnp.transpose(rwt_ref[...])

    x = x_ref[...].astype(jnp.bfloat16)
    dn_t = (((1,), (1,)), ((), ()))
    dn_n = (((1,), (0,)), ((), ()))
    w0e = w0_ref[0].astype(jnp.bfloat16)
    w1e = w1_ref[0].astype(jnp.bfloat16)
    w2e = w2t_ref[0].astype(jnp.bfloat16)
    g = jax.lax.dot_general(x, w0e, dn_t, preferred_element_type=jnp.float32)
    g = g * s0_ref[e]
    u = jax.lax.dot_general(x, w1e, dn_t, preferred_element_type=jnp.float32)
    h = ((g * jax.nn.sigmoid(g)) * u).astype(jnp.bfloat16)
    y = jax.lax.dot_general(h, w2e, dn_n, preferred_element_type=jnp.float32)
    cw = jnp.sum(jnp.where(se_v[...] == e, rw_v[...], 0.0),
                 axis=1, keepdims=True)
    contrib = y * (cw * (s1_ref[e] * s2_ref[e]))

    @pl.when(e == 0)
    def _():
        o_ref[...] = contrib

    @pl.when(e != 0)
    def _():
        o_ref[...] += contrib


def kernel(x, w0, w1, w2, s0, s1, s2, selected_experts, routing_weights,
           gathered_experts_out_buf, select_experts_middle, routing_weights_middle,
           gather_buffer, scatter_buffer, use_ppl):
    se_t = jnp.swapaxes(selected_experts.astype(jnp.int32), 0, 1)
    rw_t = jnp.swapaxes(routing_weights, 0, 1)
    w2t = jnp.swapaxes(w2, 1, 2)
    out = pl.pallas_call(
        _moe_kernel,
        grid=(E,),
        in_specs=[
            pl.BlockSpec((K, T), lambda e: (0, 0)),
            pl.BlockSpec((K, T), lambda e: (0, 0)),
            pl.BlockSpec(memory_space=pltpu.SMEM),
            pl.BlockSpec(memory_space=pltpu.SMEM),
            pl.BlockSpec(memory_space=pltpu.SMEM),
            pl.BlockSpec((T, D), lambda e: (0, 0)),
            pl.BlockSpec((1, F, D), lambda e: (e, 0, 0)),
            pl.BlockSpec((1, F, D), lambda e: (e, 0, 0)),
            pl.BlockSpec((1, F, D), lambda e: (e, 0, 0)),
        ],
        out_specs=pl.BlockSpec((T, D), lambda e: (0, 0)),
        out_shape=jax.ShapeDtypeStruct((T, D), jnp.float32),
        scratch_shapes=[
            pltpu.VMEM((T, K), jnp.int32),
            pltpu.VMEM((T, K), jnp.float32),
        ],
    )(se_t, rw_t, s0, s1, s2, x, w0, w1, w2t)
    return out
